# baseline (speedup 1.0000x reference)
import jax
import jax.numpy as jnp
from jax import lax
from jax.experimental import pallas as pl
from jax.experimental.pallas import tpu as pltpu


def _fused_kernel(xp_hbm, w1_ref, b1_ref, w2_ref, b2_ref, w3_ref, b3_ref,
                  o_ref, xband, sem):
    n = pl.program_id(0)
    b = pl.program_id(1)
    nb = pl.num_programs(1)

    TI = xband.shape[1] - 4
    W = xband.shape[2] - 2
    cin = xband.shape[3]
    c1 = w1_ref.shape[2]
    c2 = w2_ref.shape[1]
    TO = 2 * TI
    WO = 2 * W

    def band_copy(bi, slot):
        return pltpu.make_async_copy(
            xp_hbm.at[n, pl.ds(bi * TI, TI + 4)], xband.at[slot], sem.at[slot])

    @pl.when(b == 0)
    def _():
        band_copy(0, 0).start()

    @pl.when(b + 1 < nb)
    def _():
        band_copy(b + 1, (b + 1) % 2).start()

    band_copy(b, b % 2).wait()
    xb = xband[b % 2]

    acc1 = jnp.zeros(((TI + 2) * W, c1), jnp.float32)
    for dx in range(3):
        slab = xb[:, dx:dx + W, :].reshape((TI + 4) * W, cin)
        kcat = jnp.concatenate(
            [slab[dy * W:(dy + TI + 2) * W] for dy in range(3)], axis=1)
        acc1 = acc1 + jnp.dot(kcat, w1_ref[dx],
                              preferred_element_type=jnp.float32)
    y = (acc1 + b1_ref[...]).reshape(TI + 2, W, c1)

    r_idx = lax.broadcasted_iota(jnp.int32, (TI + 2, 1, 1), 0)
    y = jnp.where((b == 0) & (r_idx == 0), y[1][None], y)
    y = jnp.where((b == nb - 1) & (r_idx == TI + 1), y[TI][None], y)
    yb = jnp.concatenate([y[:, :1], y, y[:, -1:]], axis=1)
    yb = yb.astype(jnp.bfloat16)

    ue = 0.25 * yb[:-1] + 0.75 * yb[1:]
    uo = 0.75 * yb[:-1] + 0.25 * yb[1:]
    t_idx = lax.broadcasted_iota(jnp.int32, (TI + 1, 1, 1), 0)
    uo = jnp.where((b == 0) & (t_idx == 0), 0.0, uo)
    ue = jnp.where((b == nb - 1) & (t_idx == TI), 0.0, ue)

    m_idx = lax.broadcasted_iota(jnp.int32, (1, W + 1, 1), 1)

    def colinterp(u):
        ca = 0.75 * u[:, 0:W + 1] + 0.25 * u[:, 1:W + 2]
        cb = 0.25 * u[:, 0:W + 1] + 0.75 * u[:, 1:W + 2]
        return (jnp.where(m_idx == 0, 0.0, ca),
                jnp.where(m_idx == W, 0.0, cb))

    cao, cbo = colinterp(uo)
    cae, cbe = colinterp(ue)

    fam = {("a", "o"): cao, ("b", "o"): cbo,
           ("a", "e"): cae, ("b", "e"): cbe}
    col_taps = [[("a", 0), ("b", 0), ("a", 1)],
                [("b", 0), ("a", 1), ("b", 1)]]
    row_taps = [[("o", 0), ("e", 0), ("o", 1)],
                [("e", 0), ("o", 1), ("e", 1)]]
    subs = []
    for pr in range(2):
        for pc in range(2):
            blocks = [
                fam[a, oe][toff:toff + TI, moff:moff + W, :].reshape(
                    TI * W, c1)
                for (a, moff) in col_taps[pc]
                for (oe, toff) in row_taps[pr]
            ]
            subs.append(jnp.concatenate(blocks, axis=1))
    xcat = jnp.concatenate(subs, axis=0)
    acc = jnp.dot(xcat, w2_ref[...], preferred_element_type=jnp.float32)

    acc = acc + b2_ref[...]
    acc = jnp.where(acc >= 0, acc, 0.2 * acc)
    z = acc.reshape(4, TI, W, c2)
    o4 = jnp.sum(z * w3_ref[...], axis=-1) + b3_ref[...]
    o4 = jnp.maximum(o4, 0.0)
    o_ref[0] = o4.reshape(2, 2, TI, W)


def _pack_weights(w1, b1, w2, b2, w3, b3, groups):
    cout, cin_g, _, _ = w1.shape
    cin = cin_g * groups
    cout_g = cout // groups
    wt = jnp.transpose(w1, (2, 3, 1, 0))
    dense = jnp.zeros((3, 3, cin, cout), w1.dtype)
    for g in range(groups):
        dense = dense.at[:, :, g * cin_g:(g + 1) * cin_g,
                         g * cout_g:(g + 1) * cout_g].set(
            wt[:, :, :, g * cout_g:(g + 1) * cout_g])
    w1c = jnp.stack([jnp.concatenate([dense[dy, dx] for dy in range(3)],
                                     axis=0) for dx in range(3)])
    w2t = jnp.transpose(w2, (2, 3, 1, 0))
    w2c = jnp.concatenate([w2t[dy, dx] for dx in range(3)
                           for dy in range(3)], axis=0)
    c2 = w2.shape[0]
    return (w1c.astype(jnp.bfloat16), b1.reshape(1, cout).astype(jnp.float32),
            w2c.astype(jnp.bfloat16), b2.reshape(1, c2).astype(jnp.float32),
            w3.reshape(1, 1, c2).astype(jnp.float32),
            b3.reshape(1, 1).astype(jnp.float32))


def kernel(x, w1, b1, w2, b2, w3, b3):
    N, Cin, H, W = x.shape
    c1 = w1.shape[0]
    c2 = w2.shape[0]
    w1c, b1r, w2c, b2r, w3r, b3r = _pack_weights(w1, b1, w2, b2, w3, b3,
                                                 groups=2)
    xt = jnp.transpose(x, (0, 2, 3, 1)).astype(jnp.bfloat16)
    xp = jnp.pad(xt, ((0, 0), (2, 2), (1, 1), (0, 0)))
    TI = next(t for t in (16, 8, 4, 2, 1) if H % t == 0)
    nb = H // TI
    out = pl.pallas_call(
        _fused_kernel,
        out_shape=jax.ShapeDtypeStruct((N, 2, 2, H, W), jnp.float32),
        grid=(N, nb),
        in_specs=[
            pl.BlockSpec(memory_space=pl.ANY),
            pl.BlockSpec((3, 3 * Cin, c1), lambda n, b: (0, 0, 0)),
            pl.BlockSpec((1, c1), lambda n, b: (0, 0)),
            pl.BlockSpec((9 * c1, c2), lambda n, b: (0, 0)),
            pl.BlockSpec((1, c2), lambda n, b: (0, 0)),
            pl.BlockSpec((1, 1, c2), lambda n, b: (0, 0, 0)),
            pl.BlockSpec((1, 1), lambda n, b: (0, 0)),
        ],
        out_specs=pl.BlockSpec((1, 2, 2, TI, W), lambda n, b: (n, 0, 0, b, 0)),
        scratch_shapes=[
            pltpu.VMEM((2, TI + 4, W + 2, Cin), jnp.bfloat16),
            pltpu.SemaphoreType.DMA((2,)),
        ],
        compiler_params=pltpu.CompilerParams(
            dimension_semantics=("arbitrary", "arbitrary"),
            vmem_limit_bytes=64 * 1024 * 1024),
    )(xp, w1c, b1r, w2c, b2r, w3r, b3r)
    o = jnp.transpose(out, (0, 3, 1, 4, 2)).reshape(N, 2 * H, 2 * W)
    return o[:, None, :, :]


# speedup vs baseline: 1.0202x; 1.0202x over previous
import jax
import jax.numpy as jnp
from jax import lax
from jax.experimental import pallas as pl
from jax.experimental.pallas import tpu as pltpu


def _fused_kernel(xp_hbm, w1_ref, b1_ref, w2_ref, b2_ref, w3_ref, b3_ref,
                  o_ref, xband, sem):
    n = pl.program_id(0)
    b = pl.program_id(1)
    nb = pl.num_programs(1)

    TI = xband.shape[1] - 4
    W = xband.shape[2] - 2
    cin = xband.shape[3]
    c1 = w1_ref.shape[2]
    c2 = w2_ref.shape[1]
    TO = 2 * TI
    WO = 2 * W

    def band_copy(bi, slot):
        return pltpu.make_async_copy(
            xp_hbm.at[n, pl.ds(bi * TI, TI + 4)], xband.at[slot], sem.at[slot])

    @pl.when(b == 0)
    def _():
        band_copy(0, 0).start()

    @pl.when(b + 1 < nb)
    def _():
        band_copy(b + 1, (b + 1) % 2).start()

    band_copy(b, b % 2).wait()
    xb = xband[b % 2]

    acc1 = jnp.zeros((TI * W + 2 * W, c1), jnp.float32)
    for dx in range(3):
        slab = xb[:, dx:dx + W, :].reshape((TI + 4) * W, cin)
        kcat = jnp.concatenate(
            [slab[dy * W:(dy + TI + 2) * W] for dy in range(3)], axis=1)
        acc1 = acc1 + jnp.dot(kcat, w1_ref[dx],
                              preferred_element_type=jnp.float32)
    y = (acc1 + b1_ref[...]).reshape(TI + 2, W, c1)

    r_idx = lax.broadcasted_iota(jnp.int32, (TI + 2, 1, 1), 0)
    y = jnp.where((b == 0) & (r_idx == 0), y[1][None], y)
    y = jnp.where((b == nb - 1) & (r_idx == TI + 1), y[TI][None], y)
    yb = jnp.concatenate([y[:, :1], y, y[:, -1:]], axis=1)
    yb = yb.astype(jnp.bfloat16)

    ue = 0.25 * yb[:-1] + 0.75 * yb[1:]
    uo = 0.75 * yb[:-1] + 0.25 * yb[1:]
    t_idx = lax.broadcasted_iota(jnp.int32, (TI + 1, 1, 1), 0)
    uo = jnp.where((b == 0) & (t_idx == 0), 0.0, uo)
    ue = jnp.where((b == nb - 1) & (t_idx == TI), 0.0, ue)

    m_idx = lax.broadcasted_iota(jnp.int32, (1, W + 1, 1), 1)

    def colinterp(u):
        ca = 0.75 * u[:, 0:W + 1] + 0.25 * u[:, 1:W + 2]
        cb = 0.25 * u[:, 0:W + 1] + 0.75 * u[:, 1:W + 2]
        return (jnp.where(m_idx == 0, 0.0, ca),
                jnp.where(m_idx == W, 0.0, cb))

    cao, cbo = colinterp(uo)
    cae, cbe = colinterp(ue)
    fam = {("o", "a"): cao, ("o", "b"): cbo,
           ("e", "a"): cae, ("e", "b"): cbe}

    col_taps = [[("a", 0), ("b", 0), ("a", 1)],
                [("b", 0), ("a", 1), ("b", 1)]]
    row_taps = [[("o", 0), ("e", 0), ("o", 1)],
                [("e", 0), ("o", 1), ("e", 1)]]
    subs = []
    for pr in range(2):
        for pc in range(2):
            subs.append(jnp.concatenate([
                fam[oe, a][toff:toff + TI, moff:moff + W].reshape(
                    TI * W, c1)
                for (a, moff) in col_taps[pc]
                for (oe, toff) in row_taps[pr]
            ], axis=1))
    xcat = jnp.concatenate(subs, axis=0)
    acc = jnp.dot(xcat, w2_ref[...], preferred_element_type=jnp.float32)

    acc = acc + b2_ref[...]
    acc = jnp.where(acc >= 0, acc, 0.2 * acc)
    z = acc.reshape(4, TI, W, c2)
    o4 = jnp.sum(z * w3_ref[...], axis=-1) + b3_ref[...]
    o4 = jnp.maximum(o4, 0.0)
    o_ref[0] = o4.reshape(2, 2, TI, W)


def _pack_weights(w1, b1, w2, b2, w3, b3, groups):
    cout, cin_g, _, _ = w1.shape
    cin = cin_g * groups
    cout_g = cout // groups
    wt = jnp.transpose(w1, (2, 3, 1, 0))
    dense = jnp.zeros((3, 3, cin, cout), w1.dtype)
    for g in range(groups):
        dense = dense.at[:, :, g * cin_g:(g + 1) * cin_g,
                         g * cout_g:(g + 1) * cout_g].set(
            wt[:, :, :, g * cout_g:(g + 1) * cout_g])
    w1c = jnp.stack([jnp.concatenate([dense[dy, dx] for dy in range(3)],
                                     axis=0) for dx in range(3)])
    w2t = jnp.transpose(w2, (2, 3, 1, 0))
    w2c = jnp.concatenate([w2t[dy, dx] for dx in range(3)
                           for dy in range(3)], axis=0)
    c2 = w2.shape[0]
    return (w1c.astype(jnp.bfloat16), b1.reshape(1, cout).astype(jnp.float32),
            w2c.astype(jnp.bfloat16), b2.reshape(1, c2).astype(jnp.float32),
            w3.reshape(1, 1, c2).astype(jnp.float32),
            b3.reshape(1, 1).astype(jnp.float32))


def kernel(x, w1, b1, w2, b2, w3, b3):
    N, Cin, H, W = x.shape
    c1 = w1.shape[0]
    c2 = w2.shape[0]
    w1c, b1r, w2c, b2r, w3r, b3r = _pack_weights(w1, b1, w2, b2, w3, b3,
                                                 groups=2)
    xt = jnp.transpose(x, (0, 2, 3, 1)).astype(jnp.bfloat16)
    xp = jnp.pad(xt, ((0, 0), (2, 2), (1, 1), (0, 0)))
    TI = next(t for t in (32, 16, 8, 4, 2, 1) if H % t == 0)
    nb = H // TI
    out = pl.pallas_call(
        _fused_kernel,
        out_shape=jax.ShapeDtypeStruct((N, 2, 2, H, W), jnp.float32),
        grid=(N, nb),
        in_specs=[
            pl.BlockSpec(memory_space=pl.ANY),
            pl.BlockSpec((3, 3 * Cin, c1), lambda n, b: (0, 0, 0)),
            pl.BlockSpec((1, c1), lambda n, b: (0, 0)),
            pl.BlockSpec((9 * c1, c2), lambda n, b: (0, 0)),
            pl.BlockSpec((1, c2), lambda n, b: (0, 0)),
            pl.BlockSpec((1, 1, c2), lambda n, b: (0, 0, 0)),
            pl.BlockSpec((1, 1), lambda n, b: (0, 0)),
        ],
        out_specs=pl.BlockSpec((1, 2, 2, TI, W), lambda n, b: (n, 0, 0, b, 0)),
        scratch_shapes=[
            pltpu.VMEM((2, TI + 4, W + 2, Cin), jnp.bfloat16),
            pltpu.SemaphoreType.DMA((2,)),
        ],
        compiler_params=pltpu.CompilerParams(
            dimension_semantics=("parallel", "arbitrary"),
            vmem_limit_bytes=64 * 1024 * 1024),
    )(xp, w1c, b1r, w2c, b2r, w3r, b3r)
    o = jnp.transpose(out, (0, 3, 1, 4, 2)).reshape(N, 2 * H, 2 * W)
    return o[:, None, :, :]
